# baseline (speedup 1.0000x reference)
import functools

import jax
import jax.numpy as jnp
from jax import lax
from jax.experimental import pallas as pl
from jax.experimental.pallas import tpu as pltpu
from jax.experimental.pallas import tpu_sc as plsc

F = 512
K = 8
C = 4
B = 1024
D = 64
ROW = F * C
OROW = F * K * C

NC = 2
NS = 16
NW = NC * NS
ROWS_PER_W = B // NW


def _topk_body(crd_ref, cols_ref):
    crd = crd_ref[...]
    xt = crd.T
    g = jnp.dot(xt, crd, preferred_element_type=jnp.float32)
    xx = jnp.sum(xt * xt, axis=1, keepdims=True)
    d = g * -2.0
    d = d + xx.T
    d = d + xx
    d = jnp.maximum(d, 0.0)
    d = jnp.sqrt(d)
    coliota = lax.broadcasted_iota(jnp.int32, (F, F), 1)
    c_iota = lax.broadcasted_iota(jnp.int32, (F, C), 1)
    parts = []
    for _ in range(K):
        mn = jnp.min(d, axis=1, keepdims=True)
        am = jnp.min(jnp.where(d == mn, coliota, F), axis=1, keepdims=True)
        parts.append(am * C + c_iota)
        d = jnp.where(coliota == am, jnp.inf, d)
    cols = jnp.concatenate(parts, axis=1)
    rowi = lax.broadcasted_iota(jnp.int32, (F, K * C), 0)
    ci = lax.broadcasted_iota(jnp.int32, (F, K * C), 1)
    cols_ref[...] = jnp.where((rowi == 0) & (ci < C), ci, cols)


def _topk_cols(crd):
    return pl.pallas_call(
        _topk_body,
        out_shape=jax.ShapeDtypeStruct((F, K * C), jnp.int32),
    )(crd)


def _gather_body(in_hbm, cols_hbm, out_hbm, cols_v, row_v, out_v):
    wid = lax.axis_index("s") * NC + lax.axis_index("c")
    pltpu.sync_copy(cols_hbm, cols_v)

    def row_body(r, carry):
        row = wid * ROWS_PER_W + r
        pltpu.sync_copy(in_hbm.at[row], row_v)

        def g_body(g, c2):
            base = g * 16
            colv = cols_v[pl.ds(base, 16)]
            out_v[pl.ds(base, 16)] = plsc.load_gather(row_v, [colv])
            return c2

        lax.fori_loop(0, OROW // 16, g_body, 0)
        pltpu.sync_copy(out_v, out_hbm.at[row])
        return carry

    lax.fori_loop(0, ROWS_PER_W, row_body, 0)


@jax.jit
def _gather(in2d, cols):
    mesh = plsc.VectorSubcoreMesh(core_axis_name="c", subcore_axis_name="s")
    f = functools.partial(
        pl.kernel,
        out_type=jax.ShapeDtypeStruct((B, OROW), jnp.float32),
        mesh=mesh,
        scratch_types=[
            pltpu.VMEM((OROW,), jnp.int32),
            pltpu.VMEM((ROW,), jnp.float32),
            pltpu.VMEM((OROW,), jnp.float32),
        ],
        compiler_params=pltpu.CompilerParams(needs_layout_passes=False),
    )(_gather_body)
    return f(in2d, cols)


def kernel(coordinates, inputs):
    crd = coordinates.reshape(D, F)
    cols = _topk_cols(crd).reshape(OROW)
    in2d = inputs.reshape(B, ROW)
    out2d = _gather(in2d, cols)
    return out2d.reshape(B, 1, F * K, C)


# speedup vs baseline: 1.2983x; 1.2983x over previous
import functools

import jax
import jax.numpy as jnp
from jax import lax
from jax.experimental import pallas as pl
from jax.experimental.pallas import tpu as pltpu
from jax.experimental.pallas import tpu_sc as plsc

F = 512
K = 8
C = 4
B = 1024
D = 64
ROW = F * C
OROW = F * K * C

NC = 2
NS = 16
NW = NC * NS
ROWS_PER_W = B // NW


def _topk_body(crd_ref, cols_ref):
    crd = crd_ref[...]
    xt = crd.T
    g = jnp.dot(xt, crd, preferred_element_type=jnp.float32)
    xx = jnp.sum(xt * xt, axis=1, keepdims=True)
    d = g * -2.0
    d = d + xx.T
    d = d + xx
    d = jnp.maximum(d, 0.0)
    d = jnp.sqrt(d)
    coliota = lax.broadcasted_iota(jnp.int32, (F, F), 1)
    c_iota = lax.broadcasted_iota(jnp.int32, (F, C), 1)
    parts = []
    for _ in range(K):
        mn = jnp.min(d, axis=1, keepdims=True)
        am = jnp.min(jnp.where(d == mn, coliota, F), axis=1, keepdims=True)
        parts.append(am * C + c_iota)
        d = jnp.where(coliota == am, jnp.inf, d)
    cols = jnp.concatenate(parts, axis=1)
    rowi = lax.broadcasted_iota(jnp.int32, (F, K * C), 0)
    ci = lax.broadcasted_iota(jnp.int32, (F, K * C), 1)
    cols_ref[...] = jnp.where((rowi == 0) & (ci < C), ci, cols)


def _topk_cols(crd):
    return pl.pallas_call(
        _topk_body,
        out_shape=jax.ShapeDtypeStruct((F, K * C), jnp.int32),
    )(crd)


UNROLL = 8


def _gather_body(in_hbm, cols_hbm, out_hbm, cols_v,
                 row_v0, row_v1, out_v0, out_v1,
                 insem0, insem1, outsem0, outsem1):
    wid = lax.axis_index("s") * NC + lax.axis_index("c")
    base = wid * ROWS_PER_W
    pltpu.sync_copy(cols_hbm, cols_v)

    row_v = (row_v0, row_v1)
    out_v = (out_v0, out_v1)
    insem = (insem0, insem1)
    outsem = (outsem0, outsem1)

    def in_copy(r):
        rb = r % 2
        return pltpu.make_async_copy(in_hbm.at[base + r], row_v[rb], insem[rb])

    def out_copy(r):
        rb = r % 2
        return pltpu.make_async_copy(out_v[rb], out_hbm.at[base + r], outsem[rb])

    in_copy(0).start()
    for r in range(ROWS_PER_W):
        rb = r % 2
        in_copy(r).wait()
        if r + 1 < ROWS_PER_W:
            in_copy(r + 1).start()
        if r >= 2:
            out_copy(r - 2).wait()
        src = row_v[rb]
        dst = out_v[rb]

        def g_body(g, c2, src=src, dst=dst):
            for u in range(UNROLL):
                b16 = (g * UNROLL + u) * 16
                colv = cols_v[pl.ds(b16, 16)]
                dst[pl.ds(b16, 16)] = plsc.load_gather(src, [colv])
            return c2

        lax.fori_loop(0, OROW // (16 * UNROLL), g_body, 0)
        out_copy(r).start()
    out_copy(ROWS_PER_W - 2).wait()
    out_copy(ROWS_PER_W - 1).wait()


@jax.jit
def _gather(in2d, cols):
    mesh = plsc.VectorSubcoreMesh(core_axis_name="c", subcore_axis_name="s")
    f = functools.partial(
        pl.kernel,
        out_type=jax.ShapeDtypeStruct((B, OROW), jnp.float32),
        mesh=mesh,
        scratch_types=[
            pltpu.VMEM((OROW,), jnp.int32),
            pltpu.VMEM((ROW,), jnp.float32),
            pltpu.VMEM((ROW,), jnp.float32),
            pltpu.VMEM((OROW,), jnp.float32),
            pltpu.VMEM((OROW,), jnp.float32),
            pltpu.SemaphoreType.DMA,
            pltpu.SemaphoreType.DMA,
            pltpu.SemaphoreType.DMA,
            pltpu.SemaphoreType.DMA,
        ],
        compiler_params=pltpu.CompilerParams(needs_layout_passes=False),
    )(_gather_body)
    return f(in2d, cols)


def kernel(coordinates, inputs):
    crd = coordinates.reshape(D, F)
    cols = _topk_cols(crd).reshape(OROW)
    in2d = inputs.reshape(B, ROW)
    out2d = _gather(in2d, cols)
    return out2d.reshape(B, 1, F * K, C)


# speedup vs baseline: 1.7429x; 1.3425x over previous
import functools

import jax
import jax.numpy as jnp
from jax import lax
from jax.experimental import pallas as pl
from jax.experimental.pallas import tpu as pltpu
from jax.experimental.pallas import tpu_sc as plsc

F = 512
K = 8
C = 4
B = 1024
D = 64
ROW = F * C
OROW = F * K * C

NC = 2
NS = 16
NW = NC * NS
ROWS_PER_W = B // NW


def _topk_body(crd_ref, cols_ref):
    crd = crd_ref[...]
    xt = crd.T
    g = jnp.dot(xt, crd, preferred_element_type=jnp.float32)
    xx = jnp.sum(xt * xt, axis=1, keepdims=True)
    d = g * -2.0
    d = d + xx.T
    d = d + xx
    d = jnp.maximum(d, 0.0)
    d = jnp.sqrt(d)
    coliota = lax.broadcasted_iota(jnp.int32, (F, F), 1)
    c_iota = lax.broadcasted_iota(jnp.int32, (F, C), 1)
    parts = []
    for _ in range(K):
        mn = jnp.min(d, axis=1, keepdims=True)
        am = jnp.min(jnp.where(d == mn, coliota, F), axis=1, keepdims=True)
        parts.append(am * C + c_iota)
        d = jnp.where(coliota == am, jnp.inf, d)
    cols = jnp.concatenate(parts, axis=1)
    rowi = lax.broadcasted_iota(jnp.int32, (F, K * C), 0)
    ci = lax.broadcasted_iota(jnp.int32, (F, K * C), 1)
    cols_ref[...] = jnp.where((rowi == 0) & (ci < C), ci, cols)


def _topk_cols(crd):
    return pl.pallas_call(
        _topk_body,
        out_shape=jax.ShapeDtypeStruct((F, K * C), jnp.int32),
    )(crd)


UNROLL = 8


def _gather_body(in_hbm, cols_hbm, out_hbm, cols_v,
                 row_v0, row_v1, out_v0, out_v1,
                 insem0, insem1, outsem0, outsem1):
    wid = lax.axis_index("s") * NC + lax.axis_index("c")
    base = wid * ROWS_PER_W
    pltpu.sync_copy(cols_hbm, cols_v)

    row_v = (row_v0, row_v1)
    out_v = (out_v0, out_v1)
    insem = (insem0, insem1)
    outsem = (outsem0, outsem1)

    def in_copy(r):
        rb = r % 2
        return pltpu.make_async_copy(in_hbm.at[base + r], row_v[rb], insem[rb])

    def out_copy(r):
        rb = r % 2
        return pltpu.make_async_copy(out_v[rb], out_hbm.at[base + r], outsem[rb])

    in_copy(0).start()
    for r in range(ROWS_PER_W):
        rb = r % 2
        in_copy(r).wait()
        if r + 1 < ROWS_PER_W:
            in_copy(r + 1).start()
        if r >= 2:
            out_copy(r - 2).wait()
        src = row_v[rb]
        dst = out_v[rb]

        @plsc.parallel_loop(0, OROW // 16, 1, unroll=UNROLL)
        def g_body(g, src=src, dst=dst):
            b16 = g * 16
            colv = cols_v[pl.ds(b16, 16)]
            dst[pl.ds(b16, 16)] = plsc.load_gather(src, [colv])

        out_copy(r).start()
    out_copy(ROWS_PER_W - 2).wait()
    out_copy(ROWS_PER_W - 1).wait()


@jax.jit
def _gather(in2d, cols):
    mesh = plsc.VectorSubcoreMesh(core_axis_name="c", subcore_axis_name="s")
    f = functools.partial(
        pl.kernel,
        out_type=jax.ShapeDtypeStruct((B, OROW), jnp.float32),
        mesh=mesh,
        scratch_types=[
            pltpu.VMEM((OROW,), jnp.int32),
            pltpu.VMEM((ROW,), jnp.float32),
            pltpu.VMEM((ROW,), jnp.float32),
            pltpu.VMEM((OROW,), jnp.float32),
            pltpu.VMEM((OROW,), jnp.float32),
            pltpu.SemaphoreType.DMA,
            pltpu.SemaphoreType.DMA,
            pltpu.SemaphoreType.DMA,
            pltpu.SemaphoreType.DMA,
        ],
        compiler_params=pltpu.CompilerParams(needs_layout_passes=False),
    )(_gather_body)
    return f(in2d, cols)


def kernel(coordinates, inputs):
    crd = coordinates.reshape(D, F)
    cols = _topk_cols(crd).reshape(OROW)
    in2d = inputs.reshape(B, ROW)
    out2d = _gather(in2d, cols)
    return out2d.reshape(B, 1, F * K, C)
